# Initial kernel scaffold; baseline (speedup 1.0000x reference)
import jax
import jax.numpy as jnp
from jax.experimental import pallas as pl


def kernel(x, edge_index, eps, W_gcn, b_gcn, W_mu, b_mu, W_ls, b_ls, W_d1, b_d1, W_d2, b_d2):
    raise NotImplementedError("write your pallas kernel here")


# speedup vs baseline: 2.7294x; 2.7294x over previous
import functools

import jax
import jax.numpy as jnp
from jax.experimental import pallas as pl
from jax.experimental.pallas import tpu as pltpu


def _dense_body(pre_ref, eps_ref, wg_ref, bg_ref, wmu_ref, bmu_ref,
                wls_ref, bls_ref, w1a_ref, w1b_ref, bd1_ref,
                mu_ref, ls_ref, a_ref, b_ref):
    pre = pre_ref[...]
    wg = wg_ref[...]
    enc = pre[:, 0:1] * wg[0:1, :] + pre[:, 1:2] * wg[1:2, :] + bg_ref[...]
    mu = jnp.dot(enc, wmu_ref[...], preferred_element_type=jnp.float32) + bmu_ref[...]
    ls = jnp.dot(enc, wls_ref[...], preferred_element_type=jnp.float32) + bls_ref[...]
    z = mu + eps_ref[...] * jnp.exp(0.5 * ls)
    mu_ref[...] = mu
    ls_ref[...] = ls
    a_ref[...] = jnp.dot(z, w1a_ref[...], preferred_element_type=jnp.float32) + bd1_ref[...]
    b_ref[...] = jnp.dot(z, w1b_ref[...], preferred_element_type=jnp.float32)


def _dense_stage(pre, eps, W_gcn, b_gcn, W_mu, b_mu, W_ls, b_ls, W_d1, b_d1):
    n, h = eps.shape
    BN = 2000
    grid = (n // BN,)
    w1a = W_d1[:h]
    w1b = W_d1[h:]
    row = lambda i: (i, 0)
    zero = lambda i: (0, 0)
    full = lambda shape: pl.BlockSpec(shape, zero)
    out = pl.pallas_call(
        _dense_body,
        grid=grid,
        in_specs=[
            pl.BlockSpec((BN, 2), row),
            pl.BlockSpec((BN, h), row),
            full((2, h)), full((1, h)),
            full((h, h)), full((1, h)),
            full((h, h)), full((1, h)),
            full((h, h)), full((h, h)), full((1, h)),
        ],
        out_specs=[
            pl.BlockSpec((BN, h), row),
            pl.BlockSpec((BN, h), row),
            pl.BlockSpec((BN, h), row),
            pl.BlockSpec((BN, h), row),
        ],
        out_shape=[jax.ShapeDtypeStruct((n, h), jnp.float32)] * 4,
    )(pre, eps, W_gcn, b_gcn.reshape(1, h), W_mu, b_mu.reshape(1, h),
      W_ls, b_ls.reshape(1, h), w1a, w1b, b_d1.reshape(1, h))
    return out


def _dec_body(a_ref, b_ref, w2_ref, b2_ref, out_ref):
    t = jnp.maximum(a_ref[...] + b_ref[...], 0.0)
    s = jnp.dot(t, w2_ref[...], preferred_element_type=jnp.float32) + b2_ref[0, 0]
    out_ref[...] = (1.0 / (1.0 + jnp.exp(-s))).reshape(out_ref.shape)


def _dec_stage(a_rows, b_rows, W_d2, b_d2):
    e, h = a_rows.shape
    BE = 8000
    grid = (e // BE,)
    out = pl.pallas_call(
        _dec_body,
        grid=grid,
        in_specs=[
            pl.BlockSpec((BE, h), lambda i: (i, 0)),
            pl.BlockSpec((BE, h), lambda i: (i, 0)),
            pl.BlockSpec((h, 1), lambda i: (0, 0)),
            pl.BlockSpec((1, 1), lambda i: (0, 0)),
        ],
        out_specs=pl.BlockSpec((1, 1, BE), lambda i: (i, 0, 0)),
        out_shape=jax.ShapeDtypeStruct((e // BE, 1, BE), jnp.float32),
    )(a_rows, b_rows, W_d2, b_d2.reshape(1, 1))
    return out.reshape(e)


def kernel(x, edge_index, eps, W_gcn, b_gcn, W_mu, b_mu, W_ls, b_ls, W_d1, b_d1, W_d2, b_d2):
    n = x.shape[0]
    src, dst = edge_index[0], edge_index[1]
    deg = jnp.zeros((n,), jnp.float32).at[dst].add(1.0) + 1.0
    dinv = jax.lax.rsqrt(deg)
    xs = x * dinv[:, None]
    acc = jnp.zeros((n, x.shape[1]), jnp.float32).at[dst].add(xs[src])
    pre = dinv[:, None] * acc + (dinv * dinv)[:, None] * x
    mu, logstd, a_tab, b_tab = _dense_stage(
        pre, eps, W_gcn, b_gcn, W_mu, b_mu, W_ls, b_ls, W_d1, b_d1)
    a_rows = a_tab[src]
    b_rows = b_tab[dst]
    score = _dec_stage(a_rows, b_rows, W_d2, b_d2)
    return (score, mu, logstd)


# speedup vs baseline: 9.7255x; 3.5632x over previous
import functools

import jax
import jax.numpy as jnp
from jax import lax
from jax.experimental import pallas as pl
from jax.experimental.pallas import tpu as pltpu
from jax.experimental.pallas import tpu_sc as plsc

_NTILES = 32
_SUBC = 16


def _deg_sc(dst_pad, zeros_npad):
    epad = dst_pad.shape[0]
    npad = zeros_npad.shape[0]
    ept = epad // _NTILES
    chunk = 128
    nchunks = ept // chunk
    slc = npad // _SUBC

    def body(dst_hbm, zeros_hbm, out_hbm, idx_v, ones_v, acc_sh, sem):
        c = lax.axis_index("c")
        s = lax.axis_index("s")
        wid = c * _SUBC + s
        for k in range(chunk // 16):
            ones_v[pl.ds(k * 16, 16)] = jnp.full((16,), 1.0, jnp.float32)
        pltpu.sync_copy(zeros_hbm.at[pl.ds(s * slc, slc)], acc_sh.at[pl.ds(s * slc, slc)])
        plsc.subcore_barrier()
        base = wid * ept

        def step(j, carry):
            pltpu.sync_copy(dst_hbm.at[pl.ds(base + j * chunk, chunk)], idx_v)
            pltpu.sync_copy(ones_v, acc_sh.at[idx_v], add=True)
            return carry

        lax.fori_loop(0, nchunks, step, 0)
        plsc.subcore_barrier()
        pltpu.sync_copy(acc_sh.at[pl.ds(s * slc, slc)], out_hbm.at[c, pl.ds(s * slc, slc)])

    f = pl.kernel(
        body,
        out_type=jax.ShapeDtypeStruct((2, npad), jnp.float32),
        mesh=plsc.VectorSubcoreMesh(core_axis_name="c", subcore_axis_name="s"),
        compiler_params=pltpu.CompilerParams(
            use_tc_tiling_on_sc=False, needs_layout_passes=False),
        scratch_types=[
            pltpu.VMEM((chunk,), jnp.int32),
            pltpu.VMEM((chunk,), jnp.float32),
            pltpu.VMEM_SHARED((npad,), jnp.float32),
            pltpu.SemaphoreType.DMA,
        ],
    )
    return f(dst_pad, zeros_npad)


def _msg_sc(src_pad, dst_pad, xs, zeros_npad2):
    epad = src_pad.shape[0]
    npad = zeros_npad2.shape[0]
    ept = epad // _NTILES
    chunk = 128
    nchunks = ept // chunk
    slc = npad // _SUBC

    def body(src_hbm, dst_hbm, xs_hbm, zeros_hbm, out_hbm,
             idx_s, idx_d, rows_v, acc_sh, sem):
        c = lax.axis_index("c")
        s = lax.axis_index("s")
        wid = c * _SUBC + s
        pltpu.sync_copy(zeros_hbm.at[pl.ds(s * slc, slc)], acc_sh.at[pl.ds(s * slc, slc)])
        plsc.subcore_barrier()
        base = wid * ept

        def step(j, carry):
            pltpu.sync_copy(src_hbm.at[pl.ds(base + j * chunk, chunk)], idx_s)
            pltpu.sync_copy(dst_hbm.at[pl.ds(base + j * chunk, chunk)], idx_d)
            pltpu.async_copy(xs_hbm.at[idx_s], rows_v, sem).wait()
            pltpu.sync_copy(rows_v, acc_sh.at[idx_d], add=True)
            return carry

        lax.fori_loop(0, nchunks, step, 0)
        plsc.subcore_barrier()
        pltpu.sync_copy(acc_sh.at[pl.ds(s * slc, slc)], out_hbm.at[c, pl.ds(s * slc, slc)])

    f = pl.kernel(
        body,
        out_type=jax.ShapeDtypeStruct((2, npad, 8), jnp.float32),
        mesh=plsc.VectorSubcoreMesh(core_axis_name="c", subcore_axis_name="s"),
        compiler_params=pltpu.CompilerParams(
            use_tc_tiling_on_sc=False, needs_layout_passes=False),
        scratch_types=[
            pltpu.VMEM((chunk,), jnp.int32),
            pltpu.VMEM((chunk,), jnp.int32),
            pltpu.VMEM((chunk, 8), jnp.float32),
            pltpu.VMEM_SHARED((npad, 8), jnp.float32),
            pltpu.SemaphoreType.DMA,
        ],
    )
    return f(src_pad, dst_pad, xs, zeros_npad2)


def _dec_sc(src_pad, dst_pad, a_tab, b_tab, w2_flat, b2):
    epad = src_pad.shape[0]
    h = a_tab.shape[1]
    ept = epad // _NTILES
    chunk = 128
    nchunks = ept // chunk
    ngroups = chunk // 16

    def body(src_hbm, dst_hbm, a_hbm, b_hbm, w2_hbm, b2_hbm, out_hbm,
             idx_s, idx_d, rows_v, w2_v, b2_v, score_v, sem_a, sem_b):
        c = lax.axis_index("c")
        s = lax.axis_index("s")
        wid = c * _SUBC + s
        pltpu.sync_copy(w2_hbm, w2_v)
        pltpu.sync_copy(b2_hbm, b2_v)
        base = wid * ept
        w2vecs = [w2_v[pl.ds(k * 16, 16)] for k in range(h // 16)]
        b2s = b2_v[...][0]

        def step(j, carry):
            pltpu.sync_copy(src_hbm.at[pl.ds(base + j * chunk, chunk)], idx_s)
            pltpu.sync_copy(dst_hbm.at[pl.ds(base + j * chunk, chunk)], idx_d)
            pltpu.async_copy(a_hbm.at[idx_s], rows_v, sem_a).wait()
            pltpu.async_copy(b_hbm.at[idx_d], rows_v, sem_b, add=True).wait()

            def group(g, carry2):
                row_ids = g * 16 + lax.iota(jnp.int32, 16)
                accs = [jnp.full((16,), 0.0, jnp.float32) for _ in range(4)]
                for hh in range(h):
                    col = jnp.full((16,), hh, jnp.int32)
                    v = plsc.load_gather(rows_v, [row_ids, col])
                    accs[hh % 4] = accs[hh % 4] + jnp.maximum(v, 0.0) * w2vecs[hh // 16][hh % 16]
                t = (accs[0] + accs[1]) + (accs[2] + accs[3]) + b2s
                score_v[pl.ds(g * 16, 16)] = 1.0 / (1.0 + jnp.exp(-t))
                return carry2

            lax.fori_loop(0, ngroups, group, 0)
            pltpu.sync_copy(score_v, out_hbm.at[pl.ds(base + j * chunk, chunk)])
            return carry

        lax.fori_loop(0, nchunks, step, 0)

    f = pl.kernel(
        body,
        out_type=jax.ShapeDtypeStruct((epad,), jnp.float32),
        mesh=plsc.VectorSubcoreMesh(core_axis_name="c", subcore_axis_name="s"),
        compiler_params=pltpu.CompilerParams(
            use_tc_tiling_on_sc=False, needs_layout_passes=False),
        scratch_types=[
            pltpu.VMEM((chunk,), jnp.int32),
            pltpu.VMEM((chunk,), jnp.int32),
            pltpu.VMEM((chunk, h), jnp.float32),
            pltpu.VMEM((h,), jnp.float32),
            pltpu.VMEM((16,), jnp.float32),
            pltpu.VMEM((chunk,), jnp.float32),
            pltpu.SemaphoreType.DMA,
            pltpu.SemaphoreType.DMA,
        ],
    )
    return f(src_pad, dst_pad, a_tab, b_tab, w2_flat, b2)


def _dense_body(pre_ref, eps_ref, wg_ref, bg_ref, wmu_ref, bmu_ref,
                wls_ref, bls_ref, w1a_ref, w1b_ref, bd1_ref,
                mu_ref, ls_ref, a_ref, b_ref):
    pre = pre_ref[...]
    wg = wg_ref[...]
    enc = pre[:, 0:1] * wg[0:1, :] + pre[:, 1:2] * wg[1:2, :] + bg_ref[...]
    mu = jnp.dot(enc, wmu_ref[...], preferred_element_type=jnp.float32) + bmu_ref[...]
    ls = jnp.dot(enc, wls_ref[...], preferred_element_type=jnp.float32) + bls_ref[...]
    z = mu + eps_ref[...] * jnp.exp(0.5 * ls)
    mu_ref[...] = mu
    ls_ref[...] = ls
    a_ref[...] = jnp.dot(z, w1a_ref[...], preferred_element_type=jnp.float32) + bd1_ref[...]
    b_ref[...] = jnp.dot(z, w1b_ref[...], preferred_element_type=jnp.float32)


def _dense_stage(pre, eps, W_gcn, b_gcn, W_mu, b_mu, W_ls, b_ls, W_d1, b_d1):
    n, h = eps.shape
    BN = 2000
    grid = (n // BN,)
    w1a = W_d1[:h]
    w1b = W_d1[h:]
    row = lambda i: (i, 0)
    zero = lambda i: (0, 0)
    full = lambda shape: pl.BlockSpec(shape, zero)
    out = pl.pallas_call(
        _dense_body,
        grid=grid,
        in_specs=[
            pl.BlockSpec((BN, 2), row),
            pl.BlockSpec((BN, h), row),
            full((2, h)), full((1, h)),
            full((h, h)), full((1, h)),
            full((h, h)), full((1, h)),
            full((h, h)), full((h, h)), full((1, h)),
        ],
        out_specs=[
            pl.BlockSpec((BN, h), row),
            pl.BlockSpec((BN, h), row),
            pl.BlockSpec((BN, h), row),
            pl.BlockSpec((BN, h), row),
        ],
        out_shape=[jax.ShapeDtypeStruct((n, h), jnp.float32)] * 4,
    )(pre, eps, W_gcn, b_gcn.reshape(1, h), W_mu, b_mu.reshape(1, h),
      W_ls, b_ls.reshape(1, h), w1a, w1b, b_d1.reshape(1, h))
    return out


def _dec_body(a_ref, b_ref, w2_ref, b2_ref, out_ref):
    t = jnp.maximum(a_ref[...] + b_ref[...], 0.0)
    s = jnp.dot(t, w2_ref[...], preferred_element_type=jnp.float32) + b2_ref[0, 0]
    out_ref[...] = (1.0 / (1.0 + jnp.exp(-s))).reshape(out_ref.shape)


def _dec_stage(a_rows, b_rows, W_d2, b_d2):
    e, h = a_rows.shape
    BE = 8000
    grid = (e // BE,)
    out = pl.pallas_call(
        _dec_body,
        grid=grid,
        in_specs=[
            pl.BlockSpec((BE, h), lambda i: (i, 0)),
            pl.BlockSpec((BE, h), lambda i: (i, 0)),
            pl.BlockSpec((h, 1), lambda i: (0, 0)),
            pl.BlockSpec((1, 1), lambda i: (0, 0)),
        ],
        out_specs=pl.BlockSpec((1, 1, BE), lambda i: (i, 0, 0)),
        out_shape=jax.ShapeDtypeStruct((e // BE, 1, BE), jnp.float32),
    )(a_rows, b_rows, W_d2, b_d2.reshape(1, 1))
    return out.reshape(e)


def kernel(x, edge_index, eps, W_gcn, b_gcn, W_mu, b_mu, W_ls, b_ls, W_d1, b_d1, W_d2, b_d2):
    n = x.shape[0]
    e = edge_index.shape[1]
    npad = 51200
    epad = ((e + 32 * 128 - 1) // (32 * 128)) * (32 * 128)
    src, dst = edge_index[0], edge_index[1]
    pad_e = epad - e
    dst_scat = jnp.concatenate([dst, jnp.full((pad_e,), npad - 1, jnp.int32)])
    zeros_npad = jnp.zeros((npad,), jnp.float32)
    degp = _deg_sc(dst_scat, zeros_npad)
    deg = degp[0, :n] + degp[1, :n] + 1.0
    dinv = jax.lax.rsqrt(deg)
    xs = x * dinv[:, None]
    src_g = jnp.concatenate([src, jnp.zeros((pad_e,), jnp.int32)])
    xs8 = jnp.pad(xs, ((0, 0), (0, 6)))
    accp = _msg_sc(src_g, dst_scat, xs8, jnp.zeros((npad, 8), jnp.float32))
    acc = accp[0, :n, :2] + accp[1, :n, :2]
    pre = dinv[:, None] * acc + (dinv * dinv)[:, None] * x
    mu, logstd, a_tab, b_tab = _dense_stage(
        pre, eps, W_gcn, b_gcn, W_mu, b_mu, W_ls, b_ls, W_d1, b_d1)
    dst_g = jnp.concatenate([dst, jnp.zeros((pad_e,), jnp.int32)])
    b2_pad = jnp.concatenate([b_d2, jnp.zeros((15,), jnp.float32)])
    score_pad = _dec_sc(src_g, dst_g, a_tab, b_tab, W_d2[:, 0], b2_pad)
    score = score_pad[:e]
    return (score, mu, logstd)


# speedup vs baseline: 9.8344x; 1.0112x over previous
import functools

import jax
import jax.numpy as jnp
from jax import lax
from jax.experimental import pallas as pl
from jax.experimental.pallas import tpu as pltpu
from jax.experimental.pallas import tpu_sc as plsc

_NTILES = 32
_SUBC = 16

_SC_PARAMS = dict(
    mesh=plsc.VectorSubcoreMesh(core_axis_name="c", subcore_axis_name="s"),
    compiler_params=pltpu.CompilerParams(
        use_tc_tiling_on_sc=False, needs_layout_passes=False),
)


def _deg_sc(dst3, zeros_npad):
    _, nck, chunk = dst3.shape
    npad = zeros_npad.shape[0]
    slc = npad // _SUBC

    def body(dst_hbm, zeros_hbm, out_hbm, idx_v, ones_v, acc_sh, sem):
        c = lax.axis_index("c")
        s = lax.axis_index("s")
        wid = c * _SUBC + s
        for k in range(chunk // 16):
            ones_v[pl.ds(k * 16, 16)] = jnp.full((16,), 1.0, jnp.float32)
        pltpu.sync_copy(zeros_hbm.at[pl.ds(s * slc, slc)], acc_sh.at[pl.ds(s * slc, slc)])
        pltpu.sync_copy(dst_hbm.at[wid], idx_v)
        plsc.subcore_barrier()

        def step(j, carry):
            pltpu.async_copy(ones_v, acc_sh.at[idx_v.at[j]], sem, add=True)
            return carry

        lax.fori_loop(0, nck, step, 0)

        def drain(j, carry):
            pltpu.make_async_copy(ones_v, acc_sh.at[idx_v.at[0]], sem).wait()
            return carry

        lax.fori_loop(0, nck, drain, 0)
        plsc.subcore_barrier()
        pltpu.sync_copy(acc_sh.at[pl.ds(s * slc, slc)], out_hbm.at[c, pl.ds(s * slc, slc)])

    f = pl.kernel(
        body,
        out_type=jax.ShapeDtypeStruct((2, npad), jnp.float32),
        **_SC_PARAMS,
        scratch_types=[
            pltpu.VMEM((nck, chunk), jnp.int32),
            pltpu.VMEM((chunk,), jnp.float32),
            pltpu.VMEM_SHARED((npad,), jnp.float32),
            pltpu.SemaphoreType.DMA,
        ],
    )
    return f(dst3, zeros_npad)


def _msg_sc(src3, dst3, xs8, zeros_npad8):
    _, nck, chunk = src3.shape
    npad = zeros_npad8.shape[0]
    slc = npad // _SUBC

    def body(src_hbm, dst_hbm, xs_hbm, zeros_hbm, out_hbm,
             idx_s, idx_d, rows_v, acc_sh, sem0, sem1):
        c = lax.axis_index("c")
        s = lax.axis_index("s")
        wid = c * _SUBC + s
        pltpu.sync_copy(zeros_hbm.at[pl.ds(s * slc, slc)], acc_sh.at[pl.ds(s * slc, slc)])
        pltpu.sync_copy(src_hbm.at[wid], idx_s)
        pltpu.sync_copy(dst_hbm.at[wid], idx_d)
        plsc.subcore_barrier()
        sems = (sem0, sem1)
        pltpu.async_copy(xs_hbm.at[idx_s.at[0]], rows_v.at[0], sem0)
        pltpu.async_copy(xs_hbm.at[idx_s.at[1]], rows_v.at[1], sem1)

        def pair(p, carry):
            for b in range(2):
                k = p * 2 + b
                pltpu.make_async_copy(xs_hbm.at[idx_s.at[0]], rows_v.at[b], sems[b]).wait()
                pltpu.sync_copy(rows_v.at[b], acc_sh.at[idx_d.at[k]], add=True)

                @pl.when(k + 2 < nck)
                def _():
                    pltpu.async_copy(xs_hbm.at[idx_s.at[k + 2]], rows_v.at[b], sems[b])
            return carry

        lax.fori_loop(0, nck // 2, pair, 0)
        plsc.subcore_barrier()
        pltpu.sync_copy(acc_sh.at[pl.ds(s * slc, slc)], out_hbm.at[c, pl.ds(s * slc, slc)])

    f = pl.kernel(
        body,
        out_type=jax.ShapeDtypeStruct((2, npad, 8), jnp.float32),
        **_SC_PARAMS,
        scratch_types=[
            pltpu.VMEM((nck, chunk), jnp.int32),
            pltpu.VMEM((nck, chunk), jnp.int32),
            pltpu.VMEM((2, chunk, 8), jnp.float32),
            pltpu.VMEM_SHARED((npad, 8), jnp.float32),
            pltpu.SemaphoreType.DMA,
            pltpu.SemaphoreType.DMA,
        ],
    )
    return f(src3, dst3, xs8, zeros_npad8)


def _dec_sc(src3, dst3, a_tab, b_tab, w2_flat, b2_pad):
    _, nck, chunk = src3.shape
    h = a_tab.shape[1]
    ngroups = chunk // 16

    def body(src_hbm, dst_hbm, a_hbm, b_hbm, w2_hbm, b2_hbm, out_hbm,
             idx_s, idx_d, bufa, bufb, w2_v, b2_v, scores_v,
             sa0, sa1, sb0, sb1):
        c = lax.axis_index("c")
        s = lax.axis_index("s")
        wid = c * _SUBC + s
        pltpu.sync_copy(w2_hbm, w2_v)
        pltpu.sync_copy(b2_hbm, b2_v)
        pltpu.sync_copy(src_hbm.at[wid], idx_s)
        pltpu.sync_copy(dst_hbm.at[wid], idx_d)
        w2vecs = [w2_v[pl.ds(k * 16, 16)] for k in range(h // 16)]
        b2s = b2_v[...][0]
        sas = (sa0, sa1)
        sbs = (sb0, sb1)
        pltpu.async_copy(a_hbm.at[idx_s.at[0]], bufa.at[0], sa0)
        pltpu.async_copy(b_hbm.at[idx_d.at[0]], bufb.at[0], sb0)
        pltpu.async_copy(a_hbm.at[idx_s.at[1]], bufa.at[1], sa1)
        pltpu.async_copy(b_hbm.at[idx_d.at[1]], bufb.at[1], sb1)

        def pair(p, carry):
            for b in range(2):
                k = p * 2 + b
                pltpu.make_async_copy(a_hbm.at[idx_s.at[0]], bufa.at[b], sas[b]).wait()
                pltpu.make_async_copy(b_hbm.at[idx_d.at[0]], bufb.at[b], sbs[b]).wait()

                def group(g, carry2):
                    row_ids = g * 16 + lax.iota(jnp.int32, 16)
                    accs = [jnp.full((16,), 0.0, jnp.float32) for _ in range(4)]
                    for hh in range(h):
                        col = jnp.full((16,), hh, jnp.int32)
                        va = plsc.load_gather(bufa.at[b], [row_ids, col])
                        vb = plsc.load_gather(bufb.at[b], [row_ids, col])
                        accs[hh % 4] = accs[hh % 4] + (
                            jnp.maximum(va + vb, 0.0) * w2vecs[hh // 16][hh % 16])
                    t = (accs[0] + accs[1]) + (accs[2] + accs[3]) + b2s
                    scores_v[k, pl.ds(g * 16, 16)] = 1.0 / (1.0 + jnp.exp(-t))
                    return carry2

                lax.fori_loop(0, ngroups, group, 0)

                @pl.when(k + 2 < nck)
                def _():
                    pltpu.async_copy(a_hbm.at[idx_s.at[k + 2]], bufa.at[b], sas[b])
                    pltpu.async_copy(b_hbm.at[idx_d.at[k + 2]], bufb.at[b], sbs[b])
            return carry

        lax.fori_loop(0, nck // 2, pair, 0)
        pltpu.sync_copy(scores_v, out_hbm.at[wid])

    f = pl.kernel(
        body,
        out_type=jax.ShapeDtypeStruct((_NTILES, nck, chunk), jnp.float32),
        **_SC_PARAMS,
        scratch_types=[
            pltpu.VMEM((nck, chunk), jnp.int32),
            pltpu.VMEM((nck, chunk), jnp.int32),
            pltpu.VMEM((2, chunk, h), jnp.float32),
            pltpu.VMEM((2, chunk, h), jnp.float32),
            pltpu.VMEM((h,), jnp.float32),
            pltpu.VMEM((16,), jnp.float32),
            pltpu.VMEM((nck, chunk), jnp.float32),
            pltpu.SemaphoreType.DMA,
            pltpu.SemaphoreType.DMA,
            pltpu.SemaphoreType.DMA,
            pltpu.SemaphoreType.DMA,
        ],
    )
    return f(src3, dst3, a_tab, b_tab, w2_flat, b2_pad)


def _dense_body(pre_ref, eps_ref, wg_ref, bg_ref, wmu_ref, bmu_ref,
                wls_ref, bls_ref, w1a_ref, w1b_ref, bd1_ref,
                mu_ref, ls_ref, a_ref, b_ref):
    pre = pre_ref[...]
    wg = wg_ref[...]
    enc = pre[:, 0:1] * wg[0:1, :] + pre[:, 1:2] * wg[1:2, :] + bg_ref[...]
    mu = jnp.dot(enc, wmu_ref[...], preferred_element_type=jnp.float32) + bmu_ref[...]
    ls = jnp.dot(enc, wls_ref[...], preferred_element_type=jnp.float32) + bls_ref[...]
    z = mu + eps_ref[...] * jnp.exp(0.5 * ls)
    mu_ref[...] = mu
    ls_ref[...] = ls
    a_ref[...] = jnp.dot(z, w1a_ref[...], preferred_element_type=jnp.float32) + bd1_ref[...]
    b_ref[...] = jnp.dot(z, w1b_ref[...], preferred_element_type=jnp.float32)


def _dense_stage(pre, eps, W_gcn, b_gcn, W_mu, b_mu, W_ls, b_ls, W_d1, b_d1):
    n, h = eps.shape
    BN = 2000
    grid = (n // BN,)
    w1a = W_d1[:h]
    w1b = W_d1[h:]
    row = lambda i: (i, 0)
    zero = lambda i: (0, 0)
    full = lambda shape: pl.BlockSpec(shape, zero)
    out = pl.pallas_call(
        _dense_body,
        grid=grid,
        in_specs=[
            pl.BlockSpec((BN, 2), row),
            pl.BlockSpec((BN, h), row),
            full((2, h)), full((1, h)),
            full((h, h)), full((1, h)),
            full((h, h)), full((1, h)),
            full((h, h)), full((h, h)), full((1, h)),
        ],
        out_specs=[
            pl.BlockSpec((BN, h), row),
            pl.BlockSpec((BN, h), row),
            pl.BlockSpec((BN, h), row),
            pl.BlockSpec((BN, h), row),
        ],
        out_shape=[jax.ShapeDtypeStruct((n, h), jnp.float32)] * 4,
    )(pre, eps, W_gcn, b_gcn.reshape(1, h), W_mu, b_mu.reshape(1, h),
      W_ls, b_ls.reshape(1, h), w1a, w1b, b_d1.reshape(1, h))
    return out


def kernel(x, edge_index, eps, W_gcn, b_gcn, W_mu, b_mu, W_ls, b_ls, W_d1, b_d1, W_d2, b_d2):
    n = x.shape[0]
    e = edge_index.shape[1]
    npad = 51200
    c_enc = 1568
    c_dec = 128
    epad = ((e + _NTILES * c_enc - 1) // (_NTILES * c_enc)) * (_NTILES * c_enc)
    src, dst = edge_index[0], edge_index[1]
    pad_e = epad - e
    dst_scat = jnp.concatenate([dst, jnp.full((pad_e,), npad - 1, jnp.int32)])
    src_g = jnp.concatenate([src, jnp.zeros((pad_e,), jnp.int32)])
    dst_g = jnp.concatenate([dst, jnp.zeros((pad_e,), jnp.int32)])
    dst_scat3 = dst_scat.reshape(_NTILES, epad // _NTILES // c_enc, c_enc)
    degp = _deg_sc(dst_scat3, jnp.zeros((npad,), jnp.float32))
    deg = degp[0, :n] + degp[1, :n] + 1.0
    dinv = jax.lax.rsqrt(deg)
    xs = x * dinv[:, None]
    xs8 = jnp.pad(xs, ((0, 0), (0, 6)))
    src_enc3 = src_g.reshape(_NTILES, epad // _NTILES // c_enc, c_enc)
    accp = _msg_sc(src_enc3, dst_scat3, xs8, jnp.zeros((npad, 8), jnp.float32))
    acc = accp[0, :n, :2] + accp[1, :n, :2]
    pre = dinv[:, None] * acc + (dinv * dinv)[:, None] * x
    mu, logstd, a_tab, b_tab = _dense_stage(
        pre, eps, W_gcn, b_gcn, W_mu, b_mu, W_ls, b_ls, W_d1, b_d1)
    src_dec3 = src_g.reshape(_NTILES, epad // _NTILES // c_dec, c_dec)
    dst_dec3 = dst_g.reshape(_NTILES, epad // _NTILES // c_dec, c_dec)
    b2_pad = jnp.concatenate([b_d2, jnp.zeros((15,), jnp.float32)])
    score3 = _dec_sc(src_dec3, dst_dec3, a_tab, b_tab, W_d2[:, 0], b2_pad)
    score = score3.reshape(epad)[:e]
    return (score, mu, logstd)
